# baseline (speedup 1.0000x reference)
import functools

import jax
import jax.numpy as jnp
from jax import lax
from jax.experimental import pallas as pl
from jax.experimental.pallas import tpu as pltpu
from jax.experimental.pallas import tpu_sc as plsc

_NUM_CODES = 1024
_EMBED = 256
_ROWS = 16 * 576
_GRID = 3
_BLK = _ROWS // _GRID


def _tc_body(x_ref, w_ref, idx_ref, loss_ref):
    i = pl.program_id(0)
    x = x_ref[...]
    w = w_ref[...]
    row_norm = jnp.sum(x * x, axis=1, keepdims=True)
    code_norm = jnp.sum(w * w, axis=1)
    mm2 = lax.dot_general(x, w + w, (((1,), (1,)), ((), ())),
                          preferred_element_type=jnp.float32)
    scores = (row_norm + code_norm[None, :]) - mm2
    mins = jnp.min(scores, axis=1, keepdims=True)
    iota = lax.broadcasted_iota(jnp.int32, scores.shape, 1)
    cand = jnp.where(scores == mins, iota, _NUM_CODES)
    idx_ref[0, 0, :] = jnp.min(cand, axis=1)

    part = jnp.full_like(loss_ref, jnp.sum(mins))

    @pl.when(i == 0)
    def _():
        loss_ref[...] = part

    @pl.when(i > 0)
    def _():
        loss_ref[...] = loss_ref[...] + part

    @pl.when(i == _GRID - 1)
    def _():
        loss_ref[...] = loss_ref[...] / (_ROWS * _EMBED)


def _tc_argmin(flat_x, W):
    return pl.pallas_call(
        _tc_body,
        grid=(_GRID,),
        in_specs=[
            pl.BlockSpec((_BLK, _EMBED), lambda i: (i, 0)),
            pl.BlockSpec((_NUM_CODES, _EMBED), lambda i: (0, 0)),
        ],
        out_specs=[
            pl.BlockSpec((1, 1, _BLK), lambda i: (i, 0, 0)),
            pl.BlockSpec((1, 1), lambda i: (0, 0)),
        ],
        out_shape=[
            jax.ShapeDtypeStruct((_GRID, 1, _BLK), jnp.int32),
            jax.ShapeDtypeStruct((1, 1), jnp.float32),
        ],
    )(flat_x, W)


def _sc_gather(W, idx):
    info = plsc.get_sparse_core_info()
    nc, ns = info.num_cores, info.num_subcores
    rows_per_tile = _ROWS // (nc * ns)
    mesh = plsc.VectorSubcoreMesh(core_axis_name="c", subcore_axis_name="s")

    @functools.partial(
        pl.kernel,
        mesh=mesh,
        out_type=(
            jax.ShapeDtypeStruct((_ROWS, _EMBED), jnp.float32),
            jax.ShapeDtypeStruct((_ROWS, _EMBED), jnp.float32),
        ),
        scratch_types=[
            pltpu.VMEM((rows_per_tile,), jnp.int32),
            pltpu.VMEM((rows_per_tile, _EMBED), jnp.float32),
            pltpu.SemaphoreType.DMA,
        ],
    )
    def k(w_hbm, idx_hbm, out_hbm, out2_hbm, idx_v, rows_v, sem):
        wid = lax.axis_index("s") * nc + lax.axis_index("c")
        base = wid * rows_per_tile
        pltpu.sync_copy(idx_hbm.at[pl.ds(base, rows_per_tile)], idx_v)
        pltpu.async_copy(w_hbm.at[idx_v], rows_v, sem).wait()
        pltpu.sync_copy(rows_v, out_hbm.at[pl.ds(base, rows_per_tile)])
        pltpu.sync_copy(rows_v, out2_hbm.at[pl.ds(base, rows_per_tile)])

    return k(W, idx)


def kernel(inputs, W):
    flat_x = inputs.reshape(-1, _EMBED)
    idx, loss_acc = _tc_argmin(flat_x, W)
    idx = idx.reshape(_ROWS)
    loss = loss_acc[0, 0]
    quantized, quantized_out = _sc_gather(W, idx)
    quantized = quantized.reshape(inputs.shape)
    quantized_out = quantized_out.reshape(inputs.shape)
    encoding_indices = idx.reshape(inputs.shape[:-1])
    return quantized_out, loss, loss, quantized, encoding_indices


# speedup vs baseline: 1.0267x; 1.0267x over previous
import functools

import jax
import jax.numpy as jnp
from jax import lax
from jax.experimental import pallas as pl
from jax.experimental.pallas import tpu as pltpu
from jax.experimental.pallas import tpu_sc as plsc

_NUM_CODES = 1024
_EMBED = 256
_ROWS = 16 * 576
_GRID = 2
_BLK = _ROWS // _GRID


def _tc_body(x_ref, w_ref, idx_ref, loss_ref):
    i = pl.program_id(0)
    x = x_ref[...]
    w = w_ref[...]
    row_norm = jnp.sum(x * x, axis=1, keepdims=True)
    code_norm = jnp.sum(w * w, axis=1)
    mm2 = lax.dot_general(x, w + w, (((1,), (1,)), ((), ())),
                          preferred_element_type=jnp.float32)
    scores = (row_norm + code_norm[None, :]) - mm2
    mins = jnp.min(scores, axis=1, keepdims=True)
    iota = lax.broadcasted_iota(jnp.int32, scores.shape, 1)
    cand = jnp.where(scores == mins, iota, _NUM_CODES)
    idx_ref[0, 0, :] = jnp.min(cand, axis=1)

    part = jnp.full_like(loss_ref, jnp.sum(mins))

    @pl.when(i == 0)
    def _():
        loss_ref[...] = part

    @pl.when(i > 0)
    def _():
        loss_ref[...] = loss_ref[...] + part

    @pl.when(i == _GRID - 1)
    def _():
        loss_ref[...] = loss_ref[...] / (_ROWS * _EMBED)


def _tc_argmin(flat_x, W):
    return pl.pallas_call(
        _tc_body,
        grid=(_GRID,),
        in_specs=[
            pl.BlockSpec((_BLK, _EMBED), lambda i: (i, 0)),
            pl.BlockSpec((_NUM_CODES, _EMBED), lambda i: (0, 0)),
        ],
        out_specs=[
            pl.BlockSpec((1, 1, _BLK), lambda i: (i, 0, 0)),
            pl.BlockSpec((1, 1), lambda i: (0, 0)),
        ],
        out_shape=[
            jax.ShapeDtypeStruct((_GRID, 1, _BLK), jnp.int32),
            jax.ShapeDtypeStruct((1, 1), jnp.float32),
        ],
    )(flat_x, W)


def _sc_gather(W, idx):
    info = plsc.get_sparse_core_info()
    nc, ns = info.num_cores, info.num_subcores
    rows_per_tile = _ROWS // (nc * ns)
    mesh = plsc.VectorSubcoreMesh(core_axis_name="c", subcore_axis_name="s")

    half = rows_per_tile // 2

    @functools.partial(
        pl.kernel,
        mesh=mesh,
        out_type=(
            jax.ShapeDtypeStruct((_ROWS, _EMBED), jnp.float32),
            jax.ShapeDtypeStruct((_ROWS, _EMBED), jnp.float32),
        ),
        scratch_types=[
            pltpu.VMEM((half,), jnp.int32),
            pltpu.VMEM((half,), jnp.int32),
            pltpu.VMEM((half, _EMBED), jnp.float32),
            pltpu.VMEM((half, _EMBED), jnp.float32),
            pltpu.SemaphoreType.DMA,
            pltpu.SemaphoreType.DMA,
        ],
    )
    def k(w_hbm, idx_hbm, out_hbm, out2_hbm, idx_v0, idx_v1, rows_v0, rows_v1,
          sem0, sem1):
        wid = lax.axis_index("s") * nc + lax.axis_index("c")
        base = wid * rows_per_tile
        pltpu.sync_copy(idx_hbm.at[pl.ds(base, half)], idx_v0)
        c0 = pltpu.async_copy(w_hbm.at[idx_v0], rows_v0, sem0)
        pltpu.sync_copy(idx_hbm.at[pl.ds(base + half, half)], idx_v1)
        c1 = pltpu.async_copy(w_hbm.at[idx_v1], rows_v1, sem1)
        c0.wait()
        pltpu.sync_copy(rows_v0, out_hbm.at[pl.ds(base, half)])
        pltpu.sync_copy(rows_v0, out2_hbm.at[pl.ds(base, half)])
        c1.wait()
        pltpu.sync_copy(rows_v1, out_hbm.at[pl.ds(base + half, half)])
        pltpu.sync_copy(rows_v1, out2_hbm.at[pl.ds(base + half, half)])

    return k(W, idx)


def kernel(inputs, W):
    flat_x = inputs.reshape(-1, _EMBED)
    idx, loss_acc = _tc_argmin(flat_x, W)
    idx = idx.reshape(_ROWS)
    loss = loss_acc[0, 0]
    quantized, quantized_out = _sc_gather(W, idx)
    quantized = quantized.reshape(inputs.shape)
    quantized_out = quantized_out.reshape(inputs.shape)
    encoding_indices = idx.reshape(inputs.shape[:-1])
    return quantized_out, loss, loss, quantized, encoding_indices
